# Initial kernel scaffold; baseline (speedup 1.0000x reference)
import jax
import jax.numpy as jnp
from jax.experimental import pallas as pl


def kernel(feature, edge_index, sp_dist, anchor_eid, dists_max, Wu, bu, Wv, bv, Wo, bo):
    raise NotImplementedError("write your pallas kernel here")


# speedup vs baseline: 3.5406x; 3.5406x over previous
import functools

import jax
import jax.numpy as jnp
from jax import lax
from jax.experimental import pallas as pl
from jax.experimental.pallas import tpu as pltpu
from jax.experimental.pallas import tpu_sc as plsc

_L = 16


def _lin_body(x_ref, wut_ref, bu_ref, wvt_ref, bv_ref, u_ref, v_ref):
    x = x_ref[...]
    u_ref[...] = jnp.dot(x, wut_ref[...], preferred_element_type=jnp.float32) + bu_ref[...]
    v_ref[...] = jnp.dot(x, wvt_ref[...], preferred_element_type=jnp.float32) + bv_ref[...]


def _linear_uv(feature, WuT, bu, WvT, bv):
    n, d = feature.shape
    blk = 1000
    grid = n // blk
    out = jax.ShapeDtypeStruct((n, d), jnp.float32)
    return pl.pallas_call(
        _lin_body,
        grid=(grid,),
        in_specs=[
            pl.BlockSpec((blk, d), lambda i: (i, 0)),
            pl.BlockSpec((d, d), lambda i: (0, 0)),
            pl.BlockSpec((1, d), lambda i: (0, 0)),
            pl.BlockSpec((d, d), lambda i: (0, 0)),
            pl.BlockSpec((1, d), lambda i: (0, 0)),
        ],
        out_specs=[
            pl.BlockSpec((blk, d), lambda i: (i, 0)),
            pl.BlockSpec((blk, d), lambda i: (i, 0)),
        ],
        out_shape=[out, out],
    )(feature, WuT, bu.reshape(1, d), WvT, bv.reshape(1, d))


def _make_sc_kernel(N, E, D, A):
    info = plsc.get_sparse_core_info()
    NC, NS = info.num_cores, info.num_subcores
    NW = NC * NS
    ND = D // _L
    CN = 4
    C = CN * A
    NA = N * A
    NPW = -(-N // NW)
    NCHUNK = -(-NPW // CN)

    mesh = plsc.VectorSubcoreMesh(core_axis_name="c", subcore_axis_name="s")
    f32, i32 = jnp.float32, jnp.int32

    @functools.partial(
        pl.kernel,
        out_type=(
            jax.ShapeDtypeStruct((N, A), f32),
            jax.ShapeDtypeStruct((N, D), f32),
        ),
        mesh=mesh,
        scratch_types=[
            pltpu.VMEM((C,), i32),
            pltpu.VMEM((C,), i32),
            pltpu.VMEM((C,), i32),
            pltpu.VMEM((C,), f32),
            pltpu.VMEM((C, D), f32),
            pltpu.VMEM((C, D), f32),
            pltpu.VMEM((D,), f32),
            pltpu.VMEM((_L,), f32),
            pltpu.VMEM((CN * A,), f32),
            pltpu.VMEM((CN, D), f32),
            pltpu.SemaphoreType.DMA,
            pltpu.SemaphoreType.DMA,
            pltpu.SemaphoreType.DMA,
            pltpu.SemaphoreType.DMA,
            pltpu.SemaphoreType.DMA,
        ],
        compiler_params=pltpu.CompilerParams(needs_layout_passes=False,
                                             use_tc_tiling_on_sc=False),
    )
    def sc_kernel(u_hbm, v_hbm, src_hbm, dst_hbm, spd_hbm, anc_hbm, wo_hbm,
                  bo_hbm, pos_out, str_out,
                  eid_v, src_v, dst_v, spd_v, u_v, v_v, wo_v, bo_v,
                  pos_b, str_b, sem0, sem1, sem2, sem3, sem4):
        wid = lax.axis_index("s") * NC + lax.axis_index("c")
        n0 = wid * NPW
        n_end = jnp.minimum(n0 + NPW, N)

        pltpu.sync_copy(wo_hbm, wo_v)
        pltpu.sync_copy(bo_hbm, bo_v)
        bo_vec = bo_v[pl.ds(0, _L)]
        wo_regs = [wo_v[pl.ds(d * _L, _L)] for d in range(ND)]
        zero = jnp.zeros((_L,), f32)
        last_lane = jnp.arange(_L, dtype=i32) == (_L - 1)

        def chunk_body(c, _):
            nb = n0 + c * CN
            base = jnp.minimum(nb * A, NA - C)
            off = nb * A - base
            pltpu.sync_copy(anc_hbm.at[pl.ds(base, C)], eid_v)
            cps = [pltpu.async_copy(src_hbm.at[eid_v], src_v, sem0),
                   pltpu.async_copy(dst_hbm.at[eid_v], dst_v, sem1),
                   pltpu.async_copy(spd_hbm.at[eid_v], spd_v, sem2)]
            for cp in cps:
                cp.wait()
            cpu = pltpu.async_copy(u_hbm.at[src_v], u_v, sem3)
            cpv = pltpu.async_copy(v_hbm.at[dst_v], v_v, sem4)
            cpu.wait()
            cpv.wait()

            def node_body(j, _):
                off_j = jnp.minimum(off + j * A, C - A)
                acc = [zero] * ND
                for g in range(A // _L):
                    spd16 = spd_v[pl.ds(off_j + g * _L, _L)]
                    for a2 in range(_L):
                        i = off_j + g * _L + a2
                        spd_s = spd16[a2]
                        pvec = zero
                        for d in range(ND):
                            u = u_v[i, pl.ds(d * _L, _L)]
                            v = v_v[i, pl.ds(d * _L, _L)]
                            m = jnp.maximum(u * spd_s + v, 0.0)
                            acc[d] = acc[d] + m
                            pvec = pvec + m * wo_regs[d]
                        psum = plsc.cumsum(pvec) + bo_vec
                        plsc.store_scatter(pos_b, [jnp.full((_L,), i, i32)],
                                           psum, mask=last_lane)
                for d in range(ND):
                    str_b[j, pl.ds(d * _L, _L)] = acc[d] * (1.0 / A)

                node = nb + j

                @pl.when(node < n_end)
                def _():
                    pltpu.sync_copy(str_b.at[j], str_out.at[node])
                    pltpu.sync_copy(pos_b.at[pl.ds(off_j, A)],
                                    pos_out.at[node])
                return 0

            lax.fori_loop(0, CN, node_body, 0)
            return 0

        lax.fori_loop(0, NCHUNK, chunk_body, 0)

    return sc_kernel


def kernel(feature, edge_index, sp_dist, anchor_eid, dists_max, Wu, bu, Wv, bv, Wo, bo):
    N, D = feature.shape
    E = edge_index.shape[1]
    A = dists_max.shape[1]

    u_feat, v_feat = _linear_uv(feature, Wu.T, bu, Wv.T, bv)

    src = edge_index[0]
    dst = edge_index[1]
    spd = sp_dist.reshape(E)
    wo = Wo.reshape(D)
    bo_pad = jnp.full((_L,), bo[0], dtype=jnp.float32)

    sc = _make_sc_kernel(N, E, D, A)
    pos, struct = sc(u_feat, v_feat, src, dst, spd, anchor_eid.reshape(N * A),
                     wo, bo_pad)
    return pos, struct
